# Initial kernel scaffold; baseline (speedup 1.0000x reference)
import jax
import jax.numpy as jnp
from jax.experimental import pallas as pl


def kernel(agvs, stat, params):
    raise NotImplementedError("write your pallas kernel here")


# speedup vs baseline: 76.7016x; 76.7016x over previous
import numpy as np
import jax
import jax.numpy as jnp
from jax import lax
from jax.experimental import pallas as pl
from jax.experimental.pallas import tpu as pltpu
from jax.experimental.pallas import tpu_sc as plsc

G = 100
N = G * G
EMBED = 64
HEADS = 4
NCONV = 4
FPAD = 24
NQ = 4
QN = N // NQ
QP = 2560
NITEMS = 656
BATCH = 8


def _node_indices(coords):
    ix = jnp.clip(jnp.round(coords[..., 0] * (G - 1)), 0, G - 1).astype(jnp.int32)
    iy = jnp.clip(jnp.round(coords[..., 1] * (G - 1)), 0, G - 1).astype(jnp.int32)
    return ix * G + iy


def _sc_bin_body(idx_hbm, val_hbm, col_hbm, out_hbm, idxv, valv, colv, accv):
    c = lax.axis_index("c")
    s = lax.axis_index("s")
    wid = s * 2 + c
    b = wid // NQ
    q = wid - b * NQ
    lo = q * QN

    pltpu.sync_copy(idx_hbm.at[b], idxv)
    pltpu.sync_copy(val_hbm.at[b], valv)
    pltpu.sync_copy(col_hbm, colv)

    zero = jnp.zeros((16,), jnp.float32)

    def zbody(i, carry):
        accv[pl.ds(i * 16, 16)] = zero
        return carry

    lax.fori_loop(0, FPAD * QP // 16, zbody, 0)

    lanes = lax.iota(jnp.int32, 16)

    def gbody(g, carry):
        base = g * 16
        iv = idxv[pl.ds(base, 16)]
        vv = valv[pl.ds(base, 16)]
        kv = colv[pl.ds(base, 16)]
        inr = (iv >= lo) & (iv < lo + QN)
        addr = jnp.where(inr, kv * QP + (iv - lo), 0)
        for l in range(16):
            msk = inr & (lanes == l)
            cur = plsc.load_gather(accv, (addr,), mask=msk)
            plsc.store_scatter(accv, (addr,), cur + vv, mask=msk)
        return carry

    lax.fori_loop(0, NITEMS // 16, gbody, 0)

    for k in range(FPAD):
        pltpu.sync_copy(accv.at[pl.ds(k * QP, QP)], out_hbm.at[b, k, q])


def _sc_bin(idx, val, col):
    mesh = plsc.VectorSubcoreMesh(core_axis_name="c", subcore_axis_name="s",
                                  num_cores=2, num_subcores=16)
    f = pl.kernel(
        _sc_bin_body,
        out_type=jax.ShapeDtypeStruct((BATCH, FPAD, NQ, QP), jnp.float32),
        mesh=mesh,
        scratch_types=[
            pltpu.VMEM((NITEMS,), jnp.int32),
            pltpu.VMEM((NITEMS,), jnp.float32),
            pltpu.VMEM((NITEMS,), jnp.int32),
            pltpu.VMEM((FPAD * QP,), jnp.float32),
        ],
        compiler_params=pltpu.CompilerParams(needs_layout_passes=False),
    )
    return f(idx, val, col)


def _dist_row(xy_ref):
    n = lax.broadcasted_iota(jnp.int32, (1, N), 1)
    fi = (n // G).astype(jnp.float32) * (1.0 / (G - 1))
    fj = (n % G).astype(jnp.float32) * (1.0 / (G - 1))
    cx = xy_ref[0, 0]
    cy = xy_ref[0, 1]
    return jnp.sqrt((fi - cx) ** 2 + (fj - cy) ** 2)


def _h1_t(ni_t, dist, W1t, w21t, b1t):
    pre = jnp.dot(W1t, ni_t, preferred_element_type=jnp.float32)
    pre = pre + w21t * dist + b1t
    return jax.nn.leaky_relu(pre, 0.01)


def _k1_body(ni_ref, xy_ref, W1t_ref, w21t_ref, b1t_ref, s_ref, q_ref):
    b = pl.program_id(0)
    h1 = _h1_t(ni_ref[...], _dist_row(xy_ref), W1t_ref[...], w21t_ref[...],
               b1t_ref[...])

    @pl.when(b == 0)
    def _():
        s_ref[...] = jnp.zeros_like(s_ref)
        q_ref[...] = jnp.zeros_like(q_ref)

    s_ref[...] += jnp.sum(h1, axis=0, keepdims=True)
    q_ref[...] += jnp.sum(h1 * h1, axis=0, keepdims=True)


def _k1(ni_t, xy, W1t, w21t, b1t):
    return pl.pallas_call(
        _k1_body,
        grid=(BATCH,),
        in_specs=[
            pl.BlockSpec((None, FPAD, N), lambda b: (b, 0, 0)),
            pl.BlockSpec((None, 1, 2), lambda b: (b, 0, 0), memory_space=pltpu.SMEM),
            pl.BlockSpec((128, FPAD), lambda b: (0, 0)),
            pl.BlockSpec((128, 1), lambda b: (0, 0)),
            pl.BlockSpec((128, 1), lambda b: (0, 0)),
        ],
        out_specs=[
            pl.BlockSpec((1, N), lambda b: (0, 0)),
            pl.BlockSpec((1, N), lambda b: (0, 0)),
        ],
        out_shape=[
            jax.ShapeDtypeStruct((1, N), jnp.float32),
            jax.ShapeDtypeStruct((1, N), jnp.float32),
        ],
        compiler_params=pltpu.CompilerParams(
            dimension_semantics=("arbitrary",),
        ),
    )(ni_t, xy, W1t, w21t, b1t)


def _k2_body(ni_ref, xy_ref, W1t_ref, w21t_ref, b1t_ref, st1_ref,
             W2t_ref, b2t_ref, s_ref, q_ref):
    b = pl.program_id(0)
    h1 = _h1_t(ni_ref[...], _dist_row(xy_ref), W1t_ref[...], w21t_ref[...],
               b1t_ref[...])
    h1 = (h1 - st1_ref[0:1]) * st1_ref[1:2]
    h2 = jax.nn.leaky_relu(
        jnp.dot(W2t_ref[...], h1, preferred_element_type=jnp.float32)
        + b2t_ref[...], 0.01)

    @pl.when(b == 0)
    def _():
        s_ref[...] = jnp.zeros_like(s_ref)
        q_ref[...] = jnp.zeros_like(q_ref)

    s_ref[...] += jnp.sum(h2, axis=0, keepdims=True)
    q_ref[...] += jnp.sum(h2 * h2, axis=0, keepdims=True)


def _k2(ni_t, xy, W1t, w21t, b1t, st1, W2t, b2t):
    return pl.pallas_call(
        _k2_body,
        grid=(BATCH,),
        in_specs=[
            pl.BlockSpec((None, FPAD, N), lambda b: (b, 0, 0)),
            pl.BlockSpec((None, 1, 2), lambda b: (b, 0, 0), memory_space=pltpu.SMEM),
            pl.BlockSpec((128, FPAD), lambda b: (0, 0)),
            pl.BlockSpec((128, 1), lambda b: (0, 0)),
            pl.BlockSpec((128, 1), lambda b: (0, 0)),
            pl.BlockSpec((2, N), lambda b: (0, 0)),
            pl.BlockSpec((EMBED, 128), lambda b: (0, 0)),
            pl.BlockSpec((EMBED, 1), lambda b: (0, 0)),
        ],
        out_specs=[
            pl.BlockSpec((1, N), lambda b: (0, 0)),
            pl.BlockSpec((1, N), lambda b: (0, 0)),
        ],
        out_shape=[
            jax.ShapeDtypeStruct((1, N), jnp.float32),
            jax.ShapeDtypeStruct((1, N), jnp.float32),
        ],
        compiler_params=pltpu.CompilerParams(
            dimension_semantics=("arbitrary",),
        ),
    )(ni_t, xy, W1t, w21t, b1t, st1, W2t, b2t)


_OFFS = (0, -1, 1, -G, G)


def _shift_l(a, off):
    if off == 0:
        return a
    z = jnp.zeros((a.shape[0], abs(off)), a.dtype)
    if off < 0:
        return jnp.concatenate([z, a[:, :off]], axis=1)
    return jnp.concatenate([a[:, off:], z], axis=1)


def _k3_body(ni_ref, xy_ref, W1t_ref, w21t_ref, b1t_ref, st_ref,
             W2t_ref, b2t_ref,
             wlt_ref, wrt_ref, att_ref,
             gb_ref, gnw_ref, gnb_ref, gnm_ref, ir_ref,
             out_ref):
    h1 = _h1_t(ni_ref[...], _dist_row(xy_ref), W1t_ref[...], w21t_ref[...],
               b1t_ref[...])
    h1 = (h1 - st_ref[0:1]) * st_ref[1:2]
    h2 = jax.nn.leaky_relu(
        jnp.dot(W2t_ref[...], h1, preferred_element_type=jnp.float32)
        + b2t_ref[...], 0.01)
    x = (h2 - st_ref[2:3]) * st_ref[3:4]

    nn = lax.broadcasted_iota(jnp.int32, (1, N), 1)
    fi = nn // G
    fj = nn - fi * G
    valids = (
        None,
        fj != 0,
        fj != G - 1,
        fi != 0,
        fi != G - 1,
    )

    for L in range(NCONV):
        o = None
        for h in range(HEADS):
            xl = jnp.dot(wlt_ref[L, h], x, preferred_element_type=jnp.float32)
            xr = jnp.dot(wrt_ref[L, h], x, preferred_element_type=jnp.float32)
            att = att_ref[L, h]

            lgs = []
            mx = None
            for off, vmask in zip(_OFFS, valids):
                t = jax.nn.leaky_relu(_shift_l(xl, off) + xr, 0.2)
                lg = jnp.dot(att, t, preferred_element_type=jnp.float32)
                if vmask is not None:
                    lg = jnp.where(vmask, lg, -1e30)
                lgs.append(lg)
                mx = lg if mx is None else jnp.maximum(mx, lg)
            exps = []
            den = None
            for lg in lgs:
                e = jnp.exp(lg - mx)
                exps.append(e)
                den = e if den is None else den + e
            inv = (1.0 / HEADS) / (den + 1e-16)

            for off, e in zip(_OFFS, exps):
                term = (e * inv) * _shift_l(xl, off)
                o = term if o is None else o + term

        o = o + gb_ref[L]
        mean = jnp.mean(o, axis=1, keepdims=True)
        centered = o - gnm_ref[L] * mean
        var = jnp.mean(centered * centered, axis=1, keepdims=True)
        x = gnw_ref[L] * centered / jnp.sqrt(var + 1e-5) + gnb_ref[L]

    cols = []
    for i in range(4):
        idx = ir_ref[0, i]
        sel = (nn == idx) & (idx != N - 1)
        cols.append(jnp.sum(jnp.where(sel, x, 0.0), axis=1, keepdims=True))
    out_ref[...] = jnp.concatenate(cols, axis=1)


def _k3(ni_t, xy, W1t, w21t, b1t, st, W2t, b2t,
        wlt, wrt, att, gb, gnw, gnb, gnm, ir):
    return pl.pallas_call(
        _k3_body,
        grid=(BATCH,),
        in_specs=[
            pl.BlockSpec((None, FPAD, N), lambda b: (b, 0, 0)),
            pl.BlockSpec((None, 1, 2), lambda b: (b, 0, 0), memory_space=pltpu.SMEM),
            pl.BlockSpec((128, FPAD), lambda b: (0, 0)),
            pl.BlockSpec((128, 1), lambda b: (0, 0)),
            pl.BlockSpec((128, 1), lambda b: (0, 0)),
            pl.BlockSpec((4, N), lambda b: (0, 0)),
            pl.BlockSpec((EMBED, 128), lambda b: (0, 0)),
            pl.BlockSpec((EMBED, 1), lambda b: (0, 0)),
            pl.BlockSpec((NCONV, HEADS, EMBED, EMBED), lambda b: (0, 0, 0, 0)),
            pl.BlockSpec((NCONV, HEADS, EMBED, EMBED), lambda b: (0, 0, 0, 0)),
            pl.BlockSpec((NCONV, HEADS, 1, EMBED), lambda b: (0, 0, 0, 0)),
            pl.BlockSpec((NCONV, EMBED, 1), lambda b: (0, 0, 0)),
            pl.BlockSpec((NCONV, EMBED, 1), lambda b: (0, 0, 0)),
            pl.BlockSpec((NCONV, EMBED, 1), lambda b: (0, 0, 0)),
            pl.BlockSpec((NCONV, EMBED, 1), lambda b: (0, 0, 0)),
            pl.BlockSpec((None, 1, 4), lambda b: (b, 0, 0), memory_space=pltpu.SMEM),
        ],
        out_specs=pl.BlockSpec((None, EMBED, 4), lambda b: (b, 0, 0)),
        out_shape=jax.ShapeDtypeStruct((BATCH, EMBED, 4), jnp.float32),
        compiler_params=pltpu.CompilerParams(
            dimension_semantics=("arbitrary",),
            vmem_limit_bytes=100 * 1024 * 1024,
        ),
    )(ni_t, xy, W1t, w21t, b1t, st, W2t, b2t,
      wlt, wrt, att, gb, gnw, gnb, gnm, ir)


def _goal_body(graw_ref, nig_ref, pe_ref, gW1_ref, gb1_ref, gW2_ref, gb2_ref,
               out_ref):
    pe = pe_ref[...]
    nn = lax.broadcasted_iota(jnp.int32, (N, 1), 0)
    rows = []
    for j in (0, 1):
        cols = []
        for b in range(BATCH):
            idx = nig_ref[b, j]
            cols.append(jnp.sum(jnp.where(nn == idx, pe, 0.0), axis=0,
                                keepdims=True))
        rows.append(jnp.concatenate(cols, axis=0))
    g9 = jnp.concatenate([graw_ref[...]] + rows, axis=1)
    h = jax.nn.leaky_relu(
        jnp.dot(g9, gW1_ref[...], preferred_element_type=jnp.float32)
        + gb1_ref[...], 0.01)
    m = jnp.mean(h)
    v = jnp.mean((h - m) ** 2)
    h = (h - m) / jnp.sqrt(v + 1e-5)
    h = jax.nn.leaky_relu(
        jnp.dot(h, gW2_ref[...], preferred_element_type=jnp.float32)
        + gb2_ref[...], 0.01)
    m = jnp.mean(h)
    v = jnp.mean((h - m) ** 2)
    out_ref[...] = (h - m) / jnp.sqrt(v + 1e-5)


def _goal(graw, nig, pos_emb, gW1, gb1, gW2, gb2):
    return pl.pallas_call(
        _goal_body,
        in_specs=[
            pl.BlockSpec(graw.shape, lambda: (0, 0)),
            pl.BlockSpec(nig.shape, lambda: (0, 0), memory_space=pltpu.SMEM),
            pl.BlockSpec(pos_emb.shape, lambda: (0, 0)),
            pl.BlockSpec(gW1.shape, lambda: (0, 0)),
            pl.BlockSpec(gb1.shape, lambda: (0, 0)),
            pl.BlockSpec(gW2.shape, lambda: (0, 0)),
            pl.BlockSpec(gb2.shape, lambda: (0, 0)),
        ],
        out_specs=pl.BlockSpec((BATCH, EMBED), lambda: (0, 0)),
        out_shape=jax.ShapeDtypeStruct((BATCH, EMBED), jnp.float32),
    )(graw, nig, pos_emb, gW1, gb1, gW2, gb2)


_COL_FLAT = np.concatenate([
    np.arange(10), np.tile(np.arange(10, 20), 63), np.full(16, 20)
]).astype(np.int32)


def kernel(agvs, stat, params):
    B = agvs.shape[0]
    coords = agvs[:, :, 2:16].reshape(B, -1, 7, 2)
    distance_percentage = agvs[:, :, 16:18]
    moving = agvs[:, :, 1]
    stat_coords = stat[:, :, :2].reshape(B, -1, 1, 2)
    obs_main = agvs[:, :1]

    indices = _node_indices(coords)
    indices = jnp.concatenate(
        [indices, jnp.repeat(indices[:, :, 0:1], 2, axis=-1), indices[:, :, 1:2]],
        axis=-1)
    vals = jnp.ones(indices.shape, agvs.dtype)
    vals = vals.at[:, :, 7].set(moving)
    vals = vals.at[:, :, 8].set(distance_percentage[:, :, 0])
    vals = vals.at[:, :, 9].set(distance_percentage[:, :, 1])
    stat_idx = _node_indices(stat_coords)[:, :, 0]
    idx_flat = jnp.concatenate(
        [indices[:, 0], indices[:, 1:].reshape(B, -1), stat_idx], axis=1)
    val_flat = jnp.concatenate(
        [vals[:, 0], vals[:, 1:].reshape(B, -1), jnp.ones((B, 16), agvs.dtype)],
        axis=1)
    col = jnp.asarray(_COL_FLAT)

    ni_t = _sc_bin(idx_flat, val_flat, col)[:, :, :, :QN].reshape(B, FPAD, N)

    pn = params['node']
    W1t = jnp.concatenate([pn['W1'].T, jnp.zeros((128, FPAD - 22), jnp.float32)],
                          axis=1)
    w21t = pn['W1'][21].reshape(128, 1)
    b1t = pn['b1'].reshape(128, 1)
    W2t = pn['W2'].T
    b2t = pn['b2'].reshape(EMBED, 1)
    xy = obs_main[:, 0, 6:8].reshape(B, 1, 2)

    s1, q1 = _k1(ni_t, xy, W1t, w21t, b1t)
    cnt1 = 1.0 / (B * 128)
    m1 = s1 * cnt1
    i1 = 1.0 / jnp.sqrt(q1 * cnt1 - m1 * m1 + 1e-5)
    st1 = jnp.concatenate([m1, i1], axis=0)
    s2, q2 = _k2(ni_t, xy, W1t, w21t, b1t, st1, W2t, b2t)
    cnt2 = 1.0 / (B * EMBED)
    m2 = s2 * cnt2
    i2 = 1.0 / jnp.sqrt(q2 * cnt2 - m2 * m2 + 1e-5)
    st = jnp.concatenate([m1, i1, m2, i2], axis=0)

    lays = params['layers']
    wlt = jnp.stack([lp['Wl'].T.reshape(HEADS, EMBED, EMBED) for lp in lays])
    wrt = jnp.stack([lp['Wr'].T.reshape(HEADS, EMBED, EMBED) for lp in lays])
    att = jnp.stack([lp['att'].reshape(HEADS, 1, EMBED) for lp in lays])
    gb = jnp.stack([lp['gat_bias'].reshape(EMBED, 1) for lp in lays])
    gnw = jnp.stack([lp['gn_weight'].reshape(EMBED, 1) for lp in lays])
    gnb = jnp.stack([lp['gn_bias'].reshape(EMBED, 1) for lp in lays])
    gnm = jnp.stack([lp['gn_mean_scale'].reshape(EMBED, 1) for lp in lays])

    in_reach = _node_indices(obs_main[:, :, 8:16].reshape(-1, 4, 2))
    in_reach = in_reach.reshape(B, 1, 4)

    filt = _k3(ni_t, xy, W1t, w21t, b1t, st, W2t, b2t,
               wlt, wrt, att, gb, gnw, gnb, gnm, in_reach)
    filt = jnp.swapaxes(filt, 1, 2)

    goal5 = jnp.concatenate([obs_main[:, 0, 4:8], obs_main[:, 0, 18:19]], -1)
    nig = jnp.stack([_node_indices(goal5[:, 0:2]), _node_indices(goal5[:, 2:4])],
                    axis=1)
    pg = params['goal']
    goal = _goal(goal5, nig, params['pos_emb'],
                 pg['W1'], pg['b1'].reshape(1, 128),
                 pg['W2'], pg['b2'].reshape(1, EMBED))

    return jnp.concatenate([filt.reshape(B, 4 * EMBED), goal], axis=1)
